# Initial kernel scaffold; baseline (speedup 1.0000x reference)
import jax
import jax.numpy as jnp
from jax.experimental import pallas as pl


def kernel(x, pos_table):
    raise NotImplementedError("write your pallas kernel here")


# speedup vs baseline: 1.6709x; 1.6709x over previous
import math

import jax
import jax.numpy as jnp
from jax.experimental import pallas as pl


def _body(x_ref, pos_ref, out_ref, *, scale):
    out_ref[...] = x_ref[...] * scale + pos_ref[...][None, :, :]


def kernel(x, pos_table):
    B, S, D = x.shape
    scale = math.sqrt(D)
    BS = 512
    n_seq = S // BS

    import functools

    grid = (n_seq, B)
    out = pl.pallas_call(
        functools.partial(_body, scale=scale),
        grid=grid,
        in_specs=[
            pl.BlockSpec((1, BS, D), lambda i, b: (b, i, 0)),
            pl.BlockSpec((BS, D), lambda i, b: (i, 0)),
        ],
        out_specs=pl.BlockSpec((1, BS, D), lambda i, b: (b, i, 0)),
        out_shape=jax.ShapeDtypeStruct((B, S, D), x.dtype),
    )(x, pos_table[:S])
    return out


# speedup vs baseline: 1.8443x; 1.1037x over previous
import math

import jax
import jax.numpy as jnp
from jax.experimental import pallas as pl


def _body(x_ref, pos_ref, out_ref, *, scale):
    out_ref[...] = x_ref[...] * scale + pos_ref[...][None, :, :]


def kernel(x, pos_table):
    B, S, D = x.shape
    scale = math.sqrt(D)
    BS = 1024
    n_seq = S // BS

    import functools

    grid = (n_seq, B)
    out = pl.pallas_call(
        functools.partial(_body, scale=scale),
        grid=grid,
        in_specs=[
            pl.BlockSpec((1, BS, D), lambda i, b: (b, i, 0)),
            pl.BlockSpec((BS, D), lambda i, b: (i, 0)),
        ],
        out_specs=pl.BlockSpec((1, BS, D), lambda i, b: (b, i, 0)),
        out_shape=jax.ShapeDtypeStruct((B, S, D), x.dtype),
    )(x, pos_table[:S])
    return out


# speedup vs baseline: 1.9666x; 1.0663x over previous
import math

import jax
import jax.numpy as jnp
from jax.experimental import pallas as pl


def _body(x_ref, pos_ref, out_ref, *, scale):
    out_ref[...] = x_ref[...] * scale + pos_ref[...][None, :, :]


def kernel(x, pos_table):
    B, S, D = x.shape
    scale = math.sqrt(D)
    BS = 2048
    n_seq = S // BS

    import functools

    grid = (n_seq, B)
    out = pl.pallas_call(
        functools.partial(_body, scale=scale),
        grid=grid,
        in_specs=[
            pl.BlockSpec((1, BS, D), lambda i, b: (b, i, 0)),
            pl.BlockSpec((BS, D), lambda i, b: (i, 0)),
        ],
        out_specs=pl.BlockSpec((1, BS, D), lambda i, b: (b, i, 0)),
        out_shape=jax.ShapeDtypeStruct((B, S, D), x.dtype),
    )(x, pos_table[:S])
    return out
